# Initial kernel scaffold; baseline (speedup 1.0000x reference)
import jax
import jax.numpy as jnp
from jax.experimental import pallas as pl


def kernel(x, xyz, W1, b1, gamma, beta, W2, b2):
    raise NotImplementedError("write your pallas kernel here")


# speedup vs baseline: 14.4696x; 14.4696x over previous
import functools

import jax
import jax.numpy as jnp
from jax.experimental import pallas as pl

_K = 20
_SWEEPS = 8


def _round_bf16(x):
    u = jax.lax.bitcast_convert_type(x, jnp.int32)
    lsb = jax.lax.shift_right_logical(u, 16) & 1
    r = (u + 0x7FFF + lsb) & ~0xFFFF
    return jax.lax.bitcast_convert_type(r, jnp.float32)


def _jacobi_smallest_vec(c00, c01, c02, c11, c12, c22):
    one = jnp.ones_like(c00)
    zero = jnp.zeros_like(c00)
    A = [[c00, c01, c02], [c01, c11, c12], [c02, c12, c22]]
    v = [one, zero, zero, zero, one, zero, zero, zero, one]
    for _ in range(_SWEEPS):
        for (p, q) in ((0, 2), (1, 2), (0, 1)):
            r = 3 - p - q
            app, aqq, apq = A[p][p], A[q][q], A[p][q]
            nz = jnp.abs(apq) > 0.0
            safe = jnp.where(nz, apq, one)
            tau = (aqq - app) / (2.0 * safe)
            sg = jnp.where(tau >= 0.0, one, -one)
            t = sg / (jnp.abs(tau) + jnp.sqrt(1.0 + tau * tau))
            t = jnp.where(nz, t, zero)
            c = 1.0 / jnp.sqrt(1.0 + t * t)
            s = t * c
            arp, arq = A[r][p], A[r][q]
            A[p][p] = app - t * apq
            A[q][q] = aqq + t * apq
            A[p][q] = zero
            A[q][p] = zero
            narp = c * arp - s * arq
            narq = s * arp + c * arq
            A[r][p] = narp
            A[p][r] = narp
            A[r][q] = narq
            A[q][r] = narq
            for i in range(3):
                vip, viq = v[3 * i + p], v[3 * i + q]
                v[3 * i + p] = c * vip - s * viq
                v[3 * i + q] = s * vip + c * viq
    d0, d1, d2 = A[0][0], A[1][1], A[2][2]
    use0 = (d0 <= d1) & (d0 <= d2)
    use1 = jnp.logical_not(use0) & (d1 <= d2)

    def pick(i):
        return jnp.where(use0, v[3 * i + 0], jnp.where(use1, v[3 * i + 1], v[3 * i + 2]))

    return pick(0), pick(1), pick(2)


def _normals_kernel(xyz_ref, xyzT_ref, qT_ref, psC_ref, qsR_ref, out_ref):
    P = xyz_ref[0]
    xyzT = xyzT_ref[0]
    qT = qT_ref[0]
    ps = psC_ref[0]
    qs = qsR_ref[0]
    E = jax.lax.dot_general(P, qT, (((1,), (0,)), ((), ())))
    D = (-2.0 * E + qs) + ps

    inf = jnp.float32(jnp.inf)
    ibig = jnp.int32(2 ** 30)
    iota = jax.lax.broadcasted_iota(jnp.int32, D.shape, 0)
    alive = jnp.ones(D.shape, jnp.bool_)
    qx, qy, qz = qT[0:1], qT[1:2], qT[2:3]
    zero = jnp.zeros_like(qs)
    prods = {e: [zero] for e in range(6)}
    m0 = jnp.min(D, axis=0, keepdims=True)
    idx0 = jnp.min(jnp.where(D == m0, iota, ibig), axis=0, keepdims=True)
    alive = jnp.logical_and(alive, iota != idx0)
    for t in range(1, _K):
        cand = jnp.where(alive, D, inf)
        m = jnp.min(cand, axis=0, keepdims=True)
        tie = cand == m
        idxv = jnp.min(jnp.where(tie, iota, ibig), axis=0, keepdims=True)
        oh_b = iota == idxv
        alive = jnp.logical_and(alive, jnp.logical_not(oh_b))
        oh = oh_b.astype(jnp.float32)
        nbrT = jax.lax.dot_general(xyzT, oh, (((1,), (0,)), ((), ())),
                                   precision=jax.lax.Precision.HIGHEST)
        rx = _round_bf16(nbrT[0:1] - qx)
        ry = _round_bf16(nbrT[1:2] - qy)
        rz = _round_bf16(nbrT[2:3] - qz)
        for e, pr in enumerate((rx * rx, ry * ry, rz * rz,
                                rx * ry, rx * rz, ry * rz)):
            prods[e].append(pr)

    def tree_sum(lst):
        arrs = lst + [zero] * (32 - len(lst))
        while len(arrs) > 1:
            arrs = [arrs[2 * i] + arrs[2 * i + 1] for i in range(len(arrs) // 2)]
        return arrs[0]

    c00, c11, c22, c01, c02, c12 = (tree_sum(prods[e]) for e in range(6))
    nx, ny, nz = _jacobi_smallest_vec(c00, c01, c02, c11, c12, c22)
    out_ref[0] = jnp.concatenate([nx, ny, nz], axis=0)


def _mlp_kernel(x_ref, nT_ref, w1x_ref, w1n_ref, b1_ref, g_ref, be_ref,
                w2_ref, b2_ref, out_ref):
    B = x_ref.shape[0]
    N = x_ref.shape[2]
    w1x = w1x_ref[...]
    w1n = w1n_ref[...]
    hs = []
    for b in range(B):
        hb = (jax.lax.dot_general(w1x, x_ref[b], (((1,), (0,)), ((), ())))
              + jax.lax.dot_general(w1n, nT_ref[b], (((1,), (0,)), ((), ()))))
        hs.append(hb)
    h = jnp.concatenate(hs, axis=1) + b1_ref[...]
    mean = jnp.mean(h, axis=1, keepdims=True)
    dev = h - mean
    var = jnp.mean(dev * dev, axis=1, keepdims=True)
    hn = (h - mean) / jnp.sqrt(var + 1e-5)
    hn = hn * g_ref[...] + be_ref[...]
    hn = jnp.maximum(hn, 0.0)
    w2 = w2_ref[...]
    b2 = b2_ref[...]
    for b in range(B):
        out_ref[b] = jax.lax.dot_general(
            w2, hn[:, b * N:(b + 1) * N], (((1,), (0,)), ((), ()))) + b2


@jax.jit
def kernel(x, xyz, W1, b1, gamma, beta, W2, b2):
    B, N, _ = xyz.shape
    C = x.shape[1]
    T = 512
    nt = N // T

    xyzT = jnp.transpose(xyz, (0, 2, 1))
    sqn = jnp.sum(xyz ** 2, axis=-1)
    sqnC = sqn[:, :, None]
    sqnR = sqn[:, None, :]

    normalsT = pl.pallas_call(
        _normals_kernel,
        grid=(B, nt),
        in_specs=[
            pl.BlockSpec((1, N, 3), lambda b, t: (b, 0, 0)),
            pl.BlockSpec((1, 3, N), lambda b, t: (b, 0, 0)),
            pl.BlockSpec((1, 3, T), lambda b, t: (b, 0, t)),
            pl.BlockSpec((1, N, 1), lambda b, t: (b, 0, 0)),
            pl.BlockSpec((1, 1, T), lambda b, t: (b, 0, t)),
        ],
        out_specs=pl.BlockSpec((1, 3, T), lambda b, t: (b, 0, t)),
        out_shape=jax.ShapeDtypeStruct((B, 3, N), jnp.float32),
    )(xyz, xyzT, xyzT, sqnC, sqnR)

    out = pl.pallas_call(
        _mlp_kernel,
        in_specs=[
            pl.BlockSpec((B, C, N), lambda: (0, 0, 0)),
            pl.BlockSpec((B, 3, N), lambda: (0, 0, 0)),
            pl.BlockSpec((C, C), lambda: (0, 0)),
            pl.BlockSpec((C, 3), lambda: (0, 0)),
            pl.BlockSpec((C, 1), lambda: (0, 0)),
            pl.BlockSpec((C, 1), lambda: (0, 0)),
            pl.BlockSpec((C, 1), lambda: (0, 0)),
            pl.BlockSpec((C, C), lambda: (0, 0)),
            pl.BlockSpec((C, 1), lambda: (0, 0)),
        ],
        out_specs=pl.BlockSpec((B, C, N), lambda: (0, 0, 0)),
        out_shape=jax.ShapeDtypeStruct((B, C, N), jnp.float32),
    )(x, normalsT, W1[:, :C], W1[:, C:], b1[:, None], gamma[:, None],
      beta[:, None], W2, b2[:, None])
    return out
